# baseline (device time: 459777 ns/iter reference)
import jax
import jax.numpy as jnp
from jax import lax
from jax.experimental import pallas as pl
from jax.experimental.pallas import tpu as pltpu

K = 8


def kernel(x):
    m, n = x.shape
    half = m // 2
    r = half // K

    def body(x_hbm, out_hbm, af, ys, yr, xs, xr,
             in_sem, om_sem, oo_sem,
             y_send, y_recv, x_send, x_recv,
             ycred, xcred):
        my_x = lax.axis_index("x")
        my_y = lax.axis_index("y")
        ynbr = (my_x, 1 - my_y)
        xnbr = (1 - my_x, my_y)

        my_base = my_x * half
        other_base = (1 - my_x) * half

        in_d, y_rd, x_rd, om_d, oo_d = [], [], [], [], []
        for c in range(K):
            s = c % 2
            in_d.append(pltpu.make_async_copy(
                x_hbm.at[pl.ds(my_base + c * r, r), :], af.at[s], in_sem.at[s]))
            y_rd.append(pltpu.make_async_remote_copy(
                src_ref=ys.at[s], dst_ref=yr.at[s],
                send_sem=y_send.at[s], recv_sem=y_recv.at[s],
                device_id=ynbr, device_id_type=pl.DeviceIdType.MESH))
            x_rd.append(pltpu.make_async_remote_copy(
                src_ref=xs.at[s], dst_ref=xr.at[s],
                send_sem=x_send.at[s], recv_sem=x_recv.at[s],
                device_id=xnbr, device_id_type=pl.DeviceIdType.MESH))
            om_d.append(pltpu.make_async_copy(
                xs.at[s], out_hbm.at[pl.ds(my_base + c * r, r), :], om_sem.at[s]))
            oo_d.append(pltpu.make_async_copy(
                xr.at[s], out_hbm.at[pl.ds(other_base + c * r, r), :],
                oo_sem.at[s]))

        in_d[0].start()
        in_d[1].start()
        in_d[0].wait()
        ys[0] = af[0].astype(jnp.bfloat16)

        barrier = pltpu.get_barrier_semaphore()
        for nbr in (ynbr, xnbr):
            pl.semaphore_signal(barrier, inc=1, device_id=nbr,
                                device_id_type=pl.DeviceIdType.MESH)
        pl.semaphore_wait(barrier, 2)

        y_rd[0].start()

        for k in range(K):
            if k + 1 < K:
                s1 = (k + 1) % 2
                in_d[k + 1].wait()
                if k >= 1:
                    y_rd[k - 1].wait_send()
                ys[s1] = af[s1].astype(jnp.bfloat16)
                if k + 1 >= 2:
                    pl.semaphore_wait(ycred, 1)
                y_rd[k + 1].start()
                if k + 2 < K:
                    in_d[k + 2].start()
            s = k % 2
            y_rd[k].wait_recv()
            if k >= 2:
                x_rd[k - 2].wait_send()
                om_d[k - 2].wait()
            xs[s] = (ys[s].astype(jnp.float32)
                     + yr[s].astype(jnp.float32)).astype(jnp.bfloat16)
            if k <= K - 3:
                pl.semaphore_signal(ycred, inc=1, device_id=ynbr,
                                    device_id_type=pl.DeviceIdType.MESH)
            om_d[k].start()
            if k >= 2:
                pl.semaphore_wait(xcred, 1)
            x_rd[k].start()
            if k >= 1:
                j = k - 1
                x_rd[j].wait_recv()
                oo_d[j].start()
                oo_d[j].wait()
                if j <= K - 3:
                    pl.semaphore_signal(xcred, inc=1, device_id=xnbr,
                                        device_id_type=pl.DeviceIdType.MESH)

        x_rd[K - 1].wait_recv()
        oo_d[K - 1].start()
        y_rd[K - 2].wait_send()
        y_rd[K - 1].wait_send()
        x_rd[K - 2].wait_send()
        x_rd[K - 1].wait_send()
        om_d[K - 2].wait()
        om_d[K - 1].wait()
        oo_d[K - 1].wait()

    return pl.pallas_call(
        body,
        out_shape=jax.ShapeDtypeStruct((m, n), jnp.bfloat16),
        in_specs=[pl.BlockSpec(memory_space=pl.ANY)],
        out_specs=pl.BlockSpec(memory_space=pl.ANY),
        scratch_shapes=[
            pltpu.VMEM((2, r, n), jnp.float32),
            pltpu.VMEM((2, r, n), jnp.bfloat16),
            pltpu.VMEM((2, r, n), jnp.bfloat16),
            pltpu.VMEM((2, r, n), jnp.bfloat16),
            pltpu.VMEM((2, r, n), jnp.bfloat16),
            pltpu.SemaphoreType.DMA((2,)),
            pltpu.SemaphoreType.DMA((2,)),
            pltpu.SemaphoreType.DMA((2,)),
            pltpu.SemaphoreType.DMA((2,)),
            pltpu.SemaphoreType.DMA((2,)),
            pltpu.SemaphoreType.DMA((2,)),
            pltpu.SemaphoreType.DMA((2,)),
            pltpu.SemaphoreType.REGULAR,
            pltpu.SemaphoreType.REGULAR,
        ],
        compiler_params=pltpu.CompilerParams(
            collective_id=0, vmem_limit_bytes=100 * 1024 * 1024),
    )(x)


# device time: 422424 ns/iter; 1.0884x vs baseline; 1.0884x over previous
import jax
import jax.numpy as jnp
from jax import lax
from jax.experimental import pallas as pl
from jax.experimental.pallas import tpu as pltpu

K = 32


def kernel(x):
    m, n = x.shape
    half = m // 2
    r = half // K

    def body(x_hbm, out_hbm, af, ys, yr, xs, xr,
             in_sem, om_sem, oo_sem,
             y_send, y_recv, x_send, x_recv,
             ycred, xcred):
        my_x = lax.axis_index("x")
        my_y = lax.axis_index("y")
        ynbr = (my_x, 1 - my_y)
        xnbr = (1 - my_x, my_y)

        my_base = my_x * half
        other_base = (1 - my_x) * half

        in_d, y_rd, x_rd, om_d, oo_d = [], [], [], [], []
        for c in range(K):
            s = c % 2
            in_d.append(pltpu.make_async_copy(
                x_hbm.at[pl.ds(my_base + c * r, r), :], af.at[s], in_sem.at[s]))
            y_rd.append(pltpu.make_async_remote_copy(
                src_ref=ys.at[s], dst_ref=yr.at[s],
                send_sem=y_send.at[s], recv_sem=y_recv.at[s],
                device_id=ynbr, device_id_type=pl.DeviceIdType.MESH))
            x_rd.append(pltpu.make_async_remote_copy(
                src_ref=xs.at[s], dst_ref=xr.at[s],
                send_sem=x_send.at[s], recv_sem=x_recv.at[s],
                device_id=xnbr, device_id_type=pl.DeviceIdType.MESH))
            om_d.append(pltpu.make_async_copy(
                xs.at[s], out_hbm.at[pl.ds(my_base + c * r, r), :], om_sem.at[s]))
            oo_d.append(pltpu.make_async_copy(
                xr.at[s], out_hbm.at[pl.ds(other_base + c * r, r), :],
                oo_sem.at[s]))

        in_d[0].start()
        in_d[1].start()
        in_d[0].wait()
        ys[0] = af[0].astype(jnp.bfloat16)

        barrier = pltpu.get_barrier_semaphore()
        for nbr in (ynbr, xnbr):
            pl.semaphore_signal(barrier, inc=1, device_id=nbr,
                                device_id_type=pl.DeviceIdType.MESH)
        pl.semaphore_wait(barrier, 2)

        y_rd[0].start()

        for k in range(K):
            if k + 1 < K:
                s1 = (k + 1) % 2
                in_d[k + 1].wait()
                if k >= 1:
                    y_rd[k - 1].wait_send()
                ys[s1] = af[s1].astype(jnp.bfloat16)
                if k + 1 >= 2:
                    pl.semaphore_wait(ycred, 1)
                y_rd[k + 1].start()
                if k + 2 < K:
                    in_d[k + 2].start()
            s = k % 2
            y_rd[k].wait_recv()
            if k >= 2:
                x_rd[k - 2].wait_send()
                om_d[k - 2].wait()
            xs[s] = (ys[s].astype(jnp.float32)
                     + yr[s].astype(jnp.float32)).astype(jnp.bfloat16)
            if k <= K - 3:
                pl.semaphore_signal(ycred, inc=1, device_id=ynbr,
                                    device_id_type=pl.DeviceIdType.MESH)
            om_d[k].start()
            if k >= 2:
                pl.semaphore_wait(xcred, 1)
            x_rd[k].start()
            if k >= 1:
                j = k - 1
                x_rd[j].wait_recv()
                oo_d[j].start()
                oo_d[j].wait()
                if j <= K - 3:
                    pl.semaphore_signal(xcred, inc=1, device_id=xnbr,
                                        device_id_type=pl.DeviceIdType.MESH)

        x_rd[K - 1].wait_recv()
        oo_d[K - 1].start()
        y_rd[K - 2].wait_send()
        y_rd[K - 1].wait_send()
        x_rd[K - 2].wait_send()
        x_rd[K - 1].wait_send()
        om_d[K - 2].wait()
        om_d[K - 1].wait()
        oo_d[K - 1].wait()

    return pl.pallas_call(
        body,
        out_shape=jax.ShapeDtypeStruct((m, n), jnp.bfloat16),
        in_specs=[pl.BlockSpec(memory_space=pl.ANY)],
        out_specs=pl.BlockSpec(memory_space=pl.ANY),
        scratch_shapes=[
            pltpu.VMEM((2, r, n), jnp.float32),
            pltpu.VMEM((2, r, n), jnp.bfloat16),
            pltpu.VMEM((2, r, n), jnp.bfloat16),
            pltpu.VMEM((2, r, n), jnp.bfloat16),
            pltpu.VMEM((2, r, n), jnp.bfloat16),
            pltpu.SemaphoreType.DMA((2,)),
            pltpu.SemaphoreType.DMA((2,)),
            pltpu.SemaphoreType.DMA((2,)),
            pltpu.SemaphoreType.DMA((2,)),
            pltpu.SemaphoreType.DMA((2,)),
            pltpu.SemaphoreType.DMA((2,)),
            pltpu.SemaphoreType.DMA((2,)),
            pltpu.SemaphoreType.REGULAR,
            pltpu.SemaphoreType.REGULAR,
        ],
        compiler_params=pltpu.CompilerParams(
            collective_id=0, vmem_limit_bytes=100 * 1024 * 1024),
    )(x)


# device time: 416411 ns/iter; 1.1041x vs baseline; 1.0144x over previous
import jax
import jax.numpy as jnp
from jax import lax
from jax.experimental import pallas as pl
from jax.experimental.pallas import tpu as pltpu

K = 64


def kernel(x):
    m, n = x.shape
    half = m // 2
    r = half // K

    def body(x_hbm, out_hbm, af, ys, yr, xs, xr,
             in_sem, om_sem, oo_sem,
             y_send, y_recv, x_send, x_recv,
             ycred, xcred):
        my_x = lax.axis_index("x")
        my_y = lax.axis_index("y")
        ynbr = (my_x, 1 - my_y)
        xnbr = (1 - my_x, my_y)

        my_base = my_x * half
        other_base = (1 - my_x) * half

        in_d, y_rd, x_rd, om_d, oo_d = [], [], [], [], []
        for c in range(K):
            s = c % 2
            in_d.append(pltpu.make_async_copy(
                x_hbm.at[pl.ds(my_base + c * r, r), :], af.at[s], in_sem.at[s]))
            y_rd.append(pltpu.make_async_remote_copy(
                src_ref=ys.at[s], dst_ref=yr.at[s],
                send_sem=y_send.at[s], recv_sem=y_recv.at[s],
                device_id=ynbr, device_id_type=pl.DeviceIdType.MESH))
            x_rd.append(pltpu.make_async_remote_copy(
                src_ref=xs.at[s], dst_ref=xr.at[s],
                send_sem=x_send.at[s], recv_sem=x_recv.at[s],
                device_id=xnbr, device_id_type=pl.DeviceIdType.MESH))
            om_d.append(pltpu.make_async_copy(
                xs.at[s], out_hbm.at[pl.ds(my_base + c * r, r), :], om_sem.at[s]))
            oo_d.append(pltpu.make_async_copy(
                xr.at[s], out_hbm.at[pl.ds(other_base + c * r, r), :],
                oo_sem.at[s]))

        in_d[0].start()
        in_d[1].start()
        in_d[0].wait()
        ys[0] = af[0].astype(jnp.bfloat16)

        barrier = pltpu.get_barrier_semaphore()
        for nbr in (ynbr, xnbr):
            pl.semaphore_signal(barrier, inc=1, device_id=nbr,
                                device_id_type=pl.DeviceIdType.MESH)
        pl.semaphore_wait(barrier, 2)

        y_rd[0].start()

        for k in range(K):
            if k + 1 < K:
                s1 = (k + 1) % 2
                in_d[k + 1].wait()
                if k >= 1:
                    y_rd[k - 1].wait_send()
                ys[s1] = af[s1].astype(jnp.bfloat16)
                if k + 1 >= 2:
                    pl.semaphore_wait(ycred, 1)
                y_rd[k + 1].start()
                if k + 2 < K:
                    in_d[k + 2].start()
            s = k % 2
            y_rd[k].wait_recv()
            if k >= 2:
                x_rd[k - 2].wait_send()
                om_d[k - 2].wait()
            xs[s] = (ys[s].astype(jnp.float32)
                     + yr[s].astype(jnp.float32)).astype(jnp.bfloat16)
            if k <= K - 3:
                pl.semaphore_signal(ycred, inc=1, device_id=ynbr,
                                    device_id_type=pl.DeviceIdType.MESH)
            om_d[k].start()
            if k >= 2:
                pl.semaphore_wait(xcred, 1)
            x_rd[k].start()
            if k >= 1:
                j = k - 1
                x_rd[j].wait_recv()
                oo_d[j].start()
                oo_d[j].wait()
                if j <= K - 3:
                    pl.semaphore_signal(xcred, inc=1, device_id=xnbr,
                                        device_id_type=pl.DeviceIdType.MESH)

        x_rd[K - 1].wait_recv()
        oo_d[K - 1].start()
        y_rd[K - 2].wait_send()
        y_rd[K - 1].wait_send()
        x_rd[K - 2].wait_send()
        x_rd[K - 1].wait_send()
        om_d[K - 2].wait()
        om_d[K - 1].wait()
        oo_d[K - 1].wait()

    return pl.pallas_call(
        body,
        out_shape=jax.ShapeDtypeStruct((m, n), jnp.bfloat16),
        in_specs=[pl.BlockSpec(memory_space=pl.ANY)],
        out_specs=pl.BlockSpec(memory_space=pl.ANY),
        scratch_shapes=[
            pltpu.VMEM((2, r, n), jnp.float32),
            pltpu.VMEM((2, r, n), jnp.bfloat16),
            pltpu.VMEM((2, r, n), jnp.bfloat16),
            pltpu.VMEM((2, r, n), jnp.bfloat16),
            pltpu.VMEM((2, r, n), jnp.bfloat16),
            pltpu.SemaphoreType.DMA((2,)),
            pltpu.SemaphoreType.DMA((2,)),
            pltpu.SemaphoreType.DMA((2,)),
            pltpu.SemaphoreType.DMA((2,)),
            pltpu.SemaphoreType.DMA((2,)),
            pltpu.SemaphoreType.DMA((2,)),
            pltpu.SemaphoreType.DMA((2,)),
            pltpu.SemaphoreType.REGULAR,
            pltpu.SemaphoreType.REGULAR,
        ],
        compiler_params=pltpu.CompilerParams(
            collective_id=0, vmem_limit_bytes=100 * 1024 * 1024),
    )(x)
